# Initial kernel scaffold; baseline (speedup 1.0000x reference)
import jax
import jax.numpy as jnp
from jax.experimental import pallas as pl


def kernel(x, edge_index, batch, W1, b1, pw1, W2, b2, pw2, W3, b3, pw3, W4, b4, lw, lb):
    raise NotImplementedError("write your pallas kernel here")


# speedup vs baseline: 14.6163x; 14.6163x over previous
import functools
import math

import jax
import jax.numpy as jnp
from jax import lax
from jax.experimental import pallas as pl
from jax.experimental.pallas import tpu as pltpu
from jax.experimental.pallas import tpu_sc as plsc

N = 10000
NPAD = 10240
E = 320000
EPAD = 327680
B = 16
NW = 32
EBLK = 80
EW = 128
RPT = NPAD // 16

_HI = lax.Precision.HIGHEST
_IMIN = -2147483648


def _dot(a, b, dims):
    return lax.dot_general(a, b, (dims, ((), ())), precision=_HI,
                           preferred_element_type=jnp.float32)


RB = 1280
NB = NPAD // RB


def _mm_body(h_ref, w_ref, o_ref):
    o_ref[...] = _dot(h_ref[...], w_ref[...], ((1,), (0,)))


def _mm(h, W):
    return pl.pallas_call(
        _mm_body,
        out_shape=jax.ShapeDtypeStruct((h.shape[0], W.shape[1]), jnp.float32),
    )(h, W)


def _mm_scaled_body(h_ref, sm_ref, w_ref, o_ref):
    o_ref[...] = _dot(sm_ref[...] * h_ref[...], w_ref[...], ((1,), (0,)))


def _mm_scaled(h, sm, W):
    return pl.pallas_call(
        _mm_scaled_body,
        out_shape=jax.ShapeDtypeStruct((h.shape[0], W.shape[1]), jnp.float32),
    )(h, sm, W)


def _prep_body(f, degp_ref, mb_ref, xw_ref, y_ref, ds_ref, dis2_ref):
    deg = 1.0 + jnp.sum(degp_ref[...], axis=1, keepdims=True)
    dis = lax.rsqrt(deg)
    ds = dis * mb_ref[:, 0:1]
    ds_ref[...] = ds
    dis2_ref[...] = dis * dis
    y_ref[:, :f] = ds * xw_ref[...]
    if f < 128:
        y_ref[:, f:] = jnp.zeros((NPAD, 128 - f), jnp.float32)


def _prep(degp_t, mb, xw):
    f = xw.shape[1]
    return pl.pallas_call(
        functools.partial(_prep_body, f),
        out_shape=(
            jax.ShapeDtypeStruct((NPAD, 128), jnp.float32),
            jax.ShapeDtypeStruct((NPAD, 1), jnp.float32),
            jax.ShapeDtypeStruct((NPAD, 1), jnp.float32),
        ),
    )(degp_t, mb, xw)


def _conv_out(aggp_ref, xw_ref, ds_ref, dis2_ref, b_ref):
    a = aggp_ref[...]
    agg = (ds_ref[...] * (a[0] + a[1]) + xw_ref[...] * dis2_ref[...]
           + b_ref[...])
    return jnp.maximum(agg, 0.0)


def _count(pred):
    return jnp.sum(jnp.where(pred, 1, 0))


def _conv_score_body(aggp_ref, xw_ref, ds_ref, dis2_ref, b_ref, pw_ref,
                     h_ref, s_ref):
    h1 = _conv_out(aggp_ref, xw_ref, ds_ref, dis2_ref, b_ref)
    h_ref[...] = h1
    s_ref[...] = _dot(h1, pw_ref[...], ((1,), (0,)))


def _conv_score(aggp, xw, ds, dis2, b, pw):
    f = xw.shape[1]
    return pl.pallas_call(
        _conv_score_body,
        grid=(NB,),
        in_specs=[
            pl.BlockSpec((2, RB, f), lambda i: (0, i, 0)),
            pl.BlockSpec((RB, f), lambda i: (i, 0)),
            pl.BlockSpec((RB, 1), lambda i: (i, 0)),
            pl.BlockSpec((RB, 1), lambda i: (i, 0)),
            pl.BlockSpec((1, f), lambda i: (0, 0)),
            pl.BlockSpec((f, 1), lambda i: (0, 0)),
        ],
        out_specs=[
            pl.BlockSpec((RB, f), lambda i: (i, 0)),
            pl.BlockSpec((RB, 1), lambda i: (i, 0)),
        ],
        out_shape=(
            jax.ShapeDtypeStruct((NPAD, f), jnp.float32),
            jax.ShapeDtypeStruct((NPAD, 1), jnp.float32),
        ),
    )(aggp, xw, ds, dis2, b.reshape(1, f), pw.reshape(f, 1))


def _select_body(k, s_ref, mb_ref, pw_ref, sm_ref, mb_out_ref):
    pw = pw_ref[...]
    pwn = jnp.sqrt(jnp.sum(pw * pw)) + 1e-16
    score = jnp.tanh(s_ref[...] / pwn)

    valid = mb_ref[:, 0:1] > 0.0
    bk = lax.bitcast_convert_type(score, jnp.int32)
    key = jnp.where(bk >= 0, bk, bk ^ jnp.int32(0x7FFFFFFF))
    key = jnp.where(valid, key, jnp.int32(_IMIN))

    nonneg = _count(key >= 0)
    lo = jnp.where(nonneg >= k, jnp.int32(0), jnp.int32(_IMIN))
    hi = jnp.where(nonneg >= k, jnp.int32(2147483647), jnp.int32(-1))

    def bis(_, lh):
        lo, hi = lh
        mid = lo + ((hi - lo) >> 1)
        down = _count(key > mid) < k
        return (jnp.where(down, lo, mid + 1), jnp.where(down, mid, hi))

    lo, _hi = lax.fori_loop(0, 31, bis, (lo, hi))
    t = lo
    need = k - _count(key > t)
    idx = lax.broadcasted_iota(jnp.int32, (NPAD, 1), 0)
    eq = key == t

    def bis2(_, lh):
        lo, hi = lh
        mid = lo + ((hi - lo) >> 1)
        down = _count(eq & (idx <= mid)) >= need
        return (jnp.where(down, lo, mid + 1), jnp.where(down, mid, hi))

    j, _ = lax.fori_loop(0, 14, bis2, (jnp.int32(0), jnp.int32(NPAD - 1)))
    sel = (key > t) | (eq & (idx <= j))
    mnew = jnp.where(sel, 1.0, 0.0)
    sm_ref[...] = score * mnew
    mb_out_ref[...] = jnp.broadcast_to(mnew, (NPAD, 16))


def _select(s, mb, pw, k):
    f = pw.shape[0]
    return pl.pallas_call(
        functools.partial(_select_body, k),
        out_shape=(
            jax.ShapeDtypeStruct((NPAD, 1), jnp.float32),
            jax.ShapeDtypeStruct((NPAD, 16), jnp.float32),
        ),
    )(s, mb, pw.reshape(f, 1))


def _final_body(aggp_ref, xw_ref, ds_ref, dis2_ref, mb_ref, bt_ref, b_ref,
                lw_ref, lb_ref, o_ref, acc_s, acc_c):
    i = pl.program_id(0)
    h1 = _conv_out(aggp_ref, xw_ref, ds_ref, dis2_ref, b_ref)
    mcol = mb_ref[:, 0:1]
    hm = h1 * mcol
    cols = lax.broadcasted_iota(jnp.int32, (RB, B), 1)
    oh = jnp.where((bt_ref[...] == cols) & (mcol > 0.0), 1.0, 0.0)
    bs = _dot(oh, hm, ((0,), (0,)))
    bc = _dot(oh, jnp.ones((RB, 1), jnp.float32), ((0,), (0,)))

    @pl.when(i == 0)
    def _():
        acc_s[...] = bs
        acc_c[...] = bc

    @pl.when(i != 0)
    def _():
        acc_s[...] += bs
        acc_c[...] += bc

    @pl.when(i == NB - 1)
    def _():
        mean = acc_s[...] / jnp.maximum(acc_c[...], 1.0)
        o_ref[...] = ((_dot(mean, lw_ref[...], ((1,), (0,))) + lb_ref[...])
                      * 100.0)


def _final(aggp, xw, ds, dis2, mb, bt, b, lw, lb):
    return pl.pallas_call(
        _final_body,
        grid=(NB,),
        in_specs=[
            pl.BlockSpec((2, RB, 16), lambda i: (0, i, 0)),
            pl.BlockSpec((RB, 16), lambda i: (i, 0)),
            pl.BlockSpec((RB, 1), lambda i: (i, 0)),
            pl.BlockSpec((RB, 1), lambda i: (i, 0)),
            pl.BlockSpec((RB, 16), lambda i: (i, 0)),
            pl.BlockSpec((RB, 1), lambda i: (i, 0)),
            pl.BlockSpec((1, 16), lambda i: (0, 0)),
            pl.BlockSpec((16, 1), lambda i: (0, 0)),
            pl.BlockSpec((1, 1), lambda i: (0, 0)),
        ],
        out_specs=pl.BlockSpec((B, 1), lambda i: (0, 0)),
        out_shape=jax.ShapeDtypeStruct((B, 1), jnp.float32),
        scratch_shapes=[
            pltpu.VMEM((B, 16), jnp.float32),
            pltpu.VMEM((B, 1), jnp.float32),
        ],
    )(aggp, xw, ds, dis2, mb, bt, b.reshape(1, 16), lw, lb.reshape(1, 1))


def _gs(y, src_b, dst_b):
    mesh = plsc.VectorSubcoreMesh(core_axis_name="c", subcore_axis_name="s")

    @functools.partial(
        pl.kernel,
        out_type=jax.ShapeDtypeStruct((2 * NPAD, 128), jnp.float32),
        mesh=mesh,
        scratch_types=[
            pltpu.VMEM((EBLK, EW), jnp.int32),
            pltpu.VMEM((EBLK, EW), jnp.int32),
            pltpu.VMEM((EW, 128), jnp.float32),
            pltpu.VMEM((16, 128), jnp.float32),
            pltpu.VMEM_SHARED((NPAD, 128), jnp.float32),
        ],
    )
    def k(y_hbm, src_hbm, dst_hbm, out_hbm, src_v, dst_v, buf, zbuf, acc_sh):
        cid = lax.axis_index("c")
        sid = lax.axis_index("s")
        wid = cid * 16 + sid

        @pl.loop(0, 16)
        def _(i):
            @pl.loop(0, 128, step=16)
            def _(j):
                zbuf[i, pl.ds(j, 16)] = jnp.zeros((16,), jnp.float32)

        @pl.loop(0, RPT, step=16)
        def _(r):
            pltpu.sync_copy(zbuf, acc_sh.at[pl.ds(sid * RPT + r, 16)])

        plsc.subcore_barrier()
        pltpu.sync_copy(src_hbm.at[wid], src_v)
        pltpu.sync_copy(dst_hbm.at[wid], dst_v)

        @pl.loop(0, EBLK)
        def _(i):
            pltpu.sync_copy(y_hbm.at[src_v.at[i]], buf)
            pltpu.sync_copy(buf, acc_sh.at[dst_v.at[i]], add=True)

        plsc.subcore_barrier()
        pltpu.sync_copy(acc_sh.at[pl.ds(sid * RPT, RPT)],
                        out_hbm.at[pl.ds(cid * NPAD + sid * RPT, RPT)])

    return k(y, src_b, dst_b).reshape(2, NPAD, 128)


EPT = EPAD // NW


def _deg_sc(m1, src_f, dst_f):
    mesh = plsc.VectorSubcoreMesh(core_axis_name="c", subcore_axis_name="s")

    @functools.partial(
        pl.kernel,
        out_type=jax.ShapeDtypeStruct((NW, NPAD), jnp.float32),
        mesh=mesh,
        compiler_params=pltpu.CompilerParams(needs_layout_passes=False),
        scratch_types=[
            pltpu.VMEM((EPT,), jnp.int32),
            pltpu.VMEM((EPT,), jnp.int32),
            pltpu.VMEM((NPAD,), jnp.float32),
            pltpu.VMEM((NPAD,), jnp.float32),
        ],
    )
    def k(m_hbm, src_hbm, dst_hbm, out_hbm, src_v, dst_v, m_v, deg_v):
        cid = lax.axis_index("c")
        sid = lax.axis_index("s")
        wid = cid * 16 + sid

        @pl.loop(0, NPAD, step=16)
        def _(i):
            deg_v[pl.ds(i, 16)] = jnp.zeros((16,), jnp.float32)

        pltpu.sync_copy(m_hbm, m_v)
        pltpu.sync_copy(src_hbm.at[wid], src_v)
        pltpu.sync_copy(dst_hbm.at[wid], dst_v)

        @pl.loop(0, EPT, step=16)
        def _(i):
            sv = src_v[pl.ds(i, 16)]
            dv = dst_v[pl.ds(i, 16)]
            mv = plsc.load_gather(m_v, [sv])
            plsc.addupdate_scatter(deg_v, [dv], mv)

        pltpu.sync_copy(deg_v, out_hbm.at[wid])

    return k(m1, src_f, dst_f)


def kernel(x, edge_index, batch, W1, b1, pw1, W2, b2, pw2, W3, b3, pw3,
           W4, b4, lw, lb):
    xp = jnp.pad(x, ((0, NPAD - N), (0, 0)))
    src = jnp.pad(edge_index[0], (0, EPAD - E), constant_values=NPAD - 1)
    dst = jnp.pad(edge_index[1], (0, EPAD - E), constant_values=NPAD - 1)
    src_b = src.reshape(NW, EBLK, EW)
    dst_b = dst.reshape(NW, EBLK, EW)
    src_f = src.reshape(NW, EPT)
    dst_f = dst.reshape(NW, EPT)
    mb = jnp.pad(jnp.ones((N, 16), jnp.float32), ((0, NPAD - N), (0, 0)))
    bt = jnp.pad(batch, (0, NPAD - N)).reshape(NPAD, 1)

    h = xp
    sm = None
    kk = N
    for W, b, pw in ((W1, b1, pw1), (W2, b2, pw2), (W3, b3, pw3)):
        kk = int(math.ceil(0.5 * kk))
        f = W.shape[1]
        xw = _mm(h, W) if sm is None else _mm_scaled(h, sm, W)
        degp = _deg_sc(mb[:, 0], src_f, dst_f)
        y, ds, dis2 = _prep(degp.T, mb, xw)
        aggp = _gs(y, src_b, dst_b)
        h, s = _conv_score(aggp[:, :, :f], xw, ds, dis2, b, pw)
        sm, mb = _select(s, mb, pw, kk)

    xw = _mm_scaled(h, sm, W4)
    degp = _deg_sc(mb[:, 0], src_f, dst_f)
    y, ds, dis2 = _prep(degp.T, mb, xw)
    aggp = _gs(y, src_b, dst_b)
    out = _final(aggp[:, :, :16], xw, ds, dis2, mb, bt, b4, lw, lb)
    return out[:, 0]


# speedup vs baseline: 15.5589x; 1.0645x over previous
import functools
import math

import jax
import jax.numpy as jnp
from jax import lax
from jax.experimental import pallas as pl
from jax.experimental.pallas import tpu as pltpu
from jax.experimental.pallas import tpu_sc as plsc

N = 10000
NPAD = 10240
E = 320000
EPAD = 327680
B = 16
NW = 32
EBLK = 80
EW = 128
EHALF = EBLK // 2
RPT = NPAD // 16

_HI = lax.Precision.HIGHEST
_IMIN = -2147483648


def _dot(a, b, dims):
    return lax.dot_general(a, b, (dims, ((), ())), precision=_HI,
                           preferred_element_type=jnp.float32)


RB = 1280
NB = NPAD // RB


def _mm_body(h_ref, w_ref, o_ref):
    o_ref[...] = _dot(h_ref[...], w_ref[...], ((1,), (0,)))


def _mm(h, W):
    return pl.pallas_call(
        _mm_body,
        out_shape=jax.ShapeDtypeStruct((h.shape[0], W.shape[1]), jnp.float32),
    )(h, W)


def _mm_scaled_body(h_ref, sm_ref, w_ref, o_ref):
    o_ref[...] = _dot(sm_ref[...] * h_ref[...], w_ref[...], ((1,), (0,)))


def _mm_scaled(h, sm, W):
    return pl.pallas_call(
        _mm_scaled_body,
        out_shape=jax.ShapeDtypeStruct((h.shape[0], W.shape[1]), jnp.float32),
    )(h, sm, W)


def _prep_body(f, degp_ref, mb_ref, xw_ref, y_ref, ds_ref, dis2_ref):
    deg = 1.0 + jnp.sum(degp_ref[...], axis=1, keepdims=True)
    dis = lax.rsqrt(deg)
    ds = dis * mb_ref[:, 0:1]
    ds_ref[...] = ds
    dis2_ref[...] = dis * dis
    y_ref[:, :f] = ds * xw_ref[...]
    if f < 128:
        y_ref[:, f:] = jnp.zeros((NPAD, 128 - f), jnp.float32)


def _prep(degp_t, mb, xw):
    f = xw.shape[1]
    return pl.pallas_call(
        functools.partial(_prep_body, f),
        out_shape=(
            jax.ShapeDtypeStruct((NPAD, 128), jnp.float32),
            jax.ShapeDtypeStruct((NPAD, 1), jnp.float32),
            jax.ShapeDtypeStruct((NPAD, 1), jnp.float32),
        ),
    )(degp_t, mb, xw)


def _conv_out(aggp_ref, xw_ref, ds_ref, dis2_ref, b_ref):
    a = aggp_ref[...]
    agg = (ds_ref[...] * (a[0] + a[1]) + xw_ref[...] * dis2_ref[...]
           + b_ref[...])
    return jnp.maximum(agg, 0.0)


def _count(pred):
    return jnp.sum(jnp.where(pred, 1, 0))


def _conv_score_body(aggp_ref, xw_ref, ds_ref, dis2_ref, b_ref, pw_ref,
                     h_ref, s_ref):
    h1 = _conv_out(aggp_ref, xw_ref, ds_ref, dis2_ref, b_ref)
    h_ref[...] = h1
    s_ref[...] = _dot(h1, pw_ref[...], ((1,), (0,)))


def _conv_score(aggp, xw, ds, dis2, b, pw):
    f = xw.shape[1]
    return pl.pallas_call(
        _conv_score_body,
        grid=(NB,),
        in_specs=[
            pl.BlockSpec((2, RB, f), lambda i: (0, i, 0)),
            pl.BlockSpec((RB, f), lambda i: (i, 0)),
            pl.BlockSpec((RB, 1), lambda i: (i, 0)),
            pl.BlockSpec((RB, 1), lambda i: (i, 0)),
            pl.BlockSpec((1, f), lambda i: (0, 0)),
            pl.BlockSpec((f, 1), lambda i: (0, 0)),
        ],
        out_specs=[
            pl.BlockSpec((RB, f), lambda i: (i, 0)),
            pl.BlockSpec((RB, 1), lambda i: (i, 0)),
        ],
        out_shape=(
            jax.ShapeDtypeStruct((NPAD, f), jnp.float32),
            jax.ShapeDtypeStruct((NPAD, 1), jnp.float32),
        ),
    )(aggp, xw, ds, dis2, b.reshape(1, f), pw.reshape(f, 1))


def _select_body(k, s_ref, mb_ref, pw_ref, sm_ref, mb_out_ref):
    pw = pw_ref[...]
    pwn = jnp.sqrt(jnp.sum(pw * pw)) + 1e-16
    score = jnp.tanh(s_ref[...] / pwn)

    valid = mb_ref[:, 0:1] > 0.0
    bk = lax.bitcast_convert_type(score, jnp.int32)
    key = jnp.where(bk >= 0, bk, bk ^ jnp.int32(0x7FFFFFFF))
    key = jnp.where(valid, key, jnp.int32(_IMIN))

    nonneg = _count(key >= 0)
    lo = jnp.where(nonneg >= k, jnp.int32(0), jnp.int32(_IMIN))
    hi = jnp.where(nonneg >= k, jnp.int32(2147483647), jnp.int32(-1))

    def bis(_, lh):
        lo, hi = lh
        mid = lo + ((hi - lo) >> 1)
        down = _count(key > mid) < k
        return (jnp.where(down, lo, mid + 1), jnp.where(down, mid, hi))

    lo, _hi = lax.fori_loop(0, 31, bis, (lo, hi))
    t = lo
    need = k - _count(key > t)
    idx = lax.broadcasted_iota(jnp.int32, (NPAD, 1), 0)
    eq = key == t

    def bis2(_, lh):
        lo, hi = lh
        mid = lo + ((hi - lo) >> 1)
        down = _count(eq & (idx <= mid)) >= need
        return (jnp.where(down, lo, mid + 1), jnp.where(down, mid, hi))

    j, _ = lax.fori_loop(0, 14, bis2, (jnp.int32(0), jnp.int32(NPAD - 1)))
    sel = (key > t) | (eq & (idx <= j))
    mnew = jnp.where(sel, 1.0, 0.0)
    sm_ref[...] = score * mnew
    mb_out_ref[...] = jnp.broadcast_to(mnew, (NPAD, 16))


def _select(s, mb, pw, k):
    f = pw.shape[0]
    return pl.pallas_call(
        functools.partial(_select_body, k),
        out_shape=(
            jax.ShapeDtypeStruct((NPAD, 1), jnp.float32),
            jax.ShapeDtypeStruct((NPAD, 16), jnp.float32),
        ),
    )(s, mb, pw.reshape(f, 1))


def _final_body(aggp_ref, xw_ref, ds_ref, dis2_ref, mb_ref, bt_ref, b_ref,
                lw_ref, lb_ref, o_ref, acc_s, acc_c):
    i = pl.program_id(0)
    h1 = _conv_out(aggp_ref, xw_ref, ds_ref, dis2_ref, b_ref)
    mcol = mb_ref[:, 0:1]
    hm = h1 * mcol
    cols = lax.broadcasted_iota(jnp.int32, (RB, B), 1)
    oh = jnp.where((bt_ref[...] == cols) & (mcol > 0.0), 1.0, 0.0)
    bs = _dot(oh, hm, ((0,), (0,)))
    bc = _dot(oh, jnp.ones((RB, 1), jnp.float32), ((0,), (0,)))

    @pl.when(i == 0)
    def _():
        acc_s[...] = bs
        acc_c[...] = bc

    @pl.when(i != 0)
    def _():
        acc_s[...] += bs
        acc_c[...] += bc

    @pl.when(i == NB - 1)
    def _():
        mean = acc_s[...] / jnp.maximum(acc_c[...], 1.0)
        o_ref[...] = ((_dot(mean, lw_ref[...], ((1,), (0,))) + lb_ref[...])
                      * 100.0)


def _final(aggp, xw, ds, dis2, mb, bt, b, lw, lb):
    return pl.pallas_call(
        _final_body,
        grid=(NB,),
        in_specs=[
            pl.BlockSpec((2, RB, 16), lambda i: (0, i, 0)),
            pl.BlockSpec((RB, 16), lambda i: (i, 0)),
            pl.BlockSpec((RB, 1), lambda i: (i, 0)),
            pl.BlockSpec((RB, 1), lambda i: (i, 0)),
            pl.BlockSpec((RB, 16), lambda i: (i, 0)),
            pl.BlockSpec((RB, 1), lambda i: (i, 0)),
            pl.BlockSpec((1, 16), lambda i: (0, 0)),
            pl.BlockSpec((16, 1), lambda i: (0, 0)),
            pl.BlockSpec((1, 1), lambda i: (0, 0)),
        ],
        out_specs=pl.BlockSpec((B, 1), lambda i: (0, 0)),
        out_shape=jax.ShapeDtypeStruct((B, 1), jnp.float32),
        scratch_shapes=[
            pltpu.VMEM((B, 16), jnp.float32),
            pltpu.VMEM((B, 1), jnp.float32),
        ],
    )(aggp, xw, ds, dis2, mb, bt, b.reshape(1, 16), lw, lb.reshape(1, 1))


def _gs(y, src_b, dst_b):
    mesh = plsc.VectorSubcoreMesh(core_axis_name="c", subcore_axis_name="s")

    @functools.partial(
        pl.kernel,
        out_type=jax.ShapeDtypeStruct((2 * NPAD, 128), jnp.float32),
        mesh=mesh,
        scratch_types=[
            pltpu.VMEM((EHALF, EW), jnp.int32),
            pltpu.VMEM((EHALF, EW), jnp.int32),
            pltpu.VMEM((EW, 128), jnp.float32),
            pltpu.VMEM((EW, 128), jnp.float32),
            pltpu.VMEM_SHARED((NPAD, 128), jnp.float32),
            pltpu.SemaphoreType.DMA,
            pltpu.SemaphoreType.DMA,
        ],
    )
    def k(y_hbm, src_hbm, dst_hbm, out_hbm, src_v, dst_v, buf0, buf1,
          acc_sh, sem0, sem1):
        cid = lax.axis_index("c")
        sid = lax.axis_index("s")
        wid = cid * 16 + sid

        @pl.loop(0, EW)
        def _(i):
            @pl.loop(0, 128, step=16)
            def _(j):
                buf0[i, pl.ds(j, 16)] = jnp.zeros((16,), jnp.float32)

        @pl.loop(0, RPT, step=EW)
        def _(r):
            pltpu.sync_copy(buf0, acc_sh.at[pl.ds(sid * RPT + r, EW)])

        plsc.subcore_barrier()

        for half in range(2):
            pltpu.sync_copy(src_hbm.at[wid, pl.ds(half * EHALF, EHALF)],
                            src_v)
            pltpu.sync_copy(dst_hbm.at[wid, pl.ds(half * EHALF, EHALF)],
                            dst_v)
            pltpu.async_copy(y_hbm.at[src_v.at[0]], buf0, sem0)

            @pl.loop(0, EHALF, step=2)
            def _(i):
                pltpu.make_async_copy(y_hbm.at[src_v.at[i]], buf0,
                                      sem0).wait()
                pltpu.async_copy(y_hbm.at[src_v.at[i + 1]], buf1, sem1)
                pltpu.sync_copy(buf0, acc_sh.at[dst_v.at[i]], add=True)
                pltpu.make_async_copy(y_hbm.at[src_v.at[i + 1]], buf1,
                                      sem1).wait()

                @pl.when(i + 2 < EHALF)
                def _():
                    pltpu.async_copy(y_hbm.at[src_v.at[i + 2]], buf0, sem0)

                pltpu.sync_copy(buf1, acc_sh.at[dst_v.at[i + 1]], add=True)

        plsc.subcore_barrier()
        pltpu.sync_copy(acc_sh.at[pl.ds(sid * RPT, RPT)],
                        out_hbm.at[pl.ds(cid * NPAD + sid * RPT, RPT)])

    return k(y, src_b, dst_b).reshape(2, NPAD, 128)


EPT = EPAD // NW


def _deg_sc(m1, src_f, dst_f):
    mesh = plsc.VectorSubcoreMesh(core_axis_name="c", subcore_axis_name="s")

    @functools.partial(
        pl.kernel,
        out_type=jax.ShapeDtypeStruct((NW, NPAD), jnp.float32),
        mesh=mesh,
        compiler_params=pltpu.CompilerParams(needs_layout_passes=False),
        scratch_types=[
            pltpu.VMEM((EPT,), jnp.int32),
            pltpu.VMEM((EPT,), jnp.int32),
            pltpu.VMEM((NPAD,), jnp.float32),
            pltpu.VMEM((NPAD,), jnp.float32),
        ],
    )
    def k(m_hbm, src_hbm, dst_hbm, out_hbm, src_v, dst_v, m_v, deg_v):
        cid = lax.axis_index("c")
        sid = lax.axis_index("s")
        wid = cid * 16 + sid

        @pl.loop(0, NPAD, step=16)
        def _(i):
            deg_v[pl.ds(i, 16)] = jnp.zeros((16,), jnp.float32)

        pltpu.sync_copy(m_hbm, m_v)
        pltpu.sync_copy(src_hbm.at[wid], src_v)
        pltpu.sync_copy(dst_hbm.at[wid], dst_v)

        @pl.loop(0, EPT, step=16)
        def _(i):
            sv = src_v[pl.ds(i, 16)]
            dv = dst_v[pl.ds(i, 16)]
            mv = plsc.load_gather(m_v, [sv])
            plsc.addupdate_scatter(deg_v, [dv], mv)

        pltpu.sync_copy(deg_v, out_hbm.at[wid])

    return k(m1, src_f, dst_f)


def kernel(x, edge_index, batch, W1, b1, pw1, W2, b2, pw2, W3, b3, pw3,
           W4, b4, lw, lb):
    xp = jnp.pad(x, ((0, NPAD - N), (0, 0)))
    src = jnp.pad(edge_index[0], (0, EPAD - E), constant_values=NPAD - 1)
    dst = jnp.pad(edge_index[1], (0, EPAD - E), constant_values=NPAD - 1)
    src_b = src.reshape(NW, EBLK, EW)
    dst_b = dst.reshape(NW, EBLK, EW)
    src_f = src.reshape(NW, EPT)
    dst_f = dst.reshape(NW, EPT)
    mb = jnp.pad(jnp.ones((N, 16), jnp.float32), ((0, NPAD - N), (0, 0)))
    bt = jnp.pad(batch, (0, NPAD - N)).reshape(NPAD, 1)

    h = xp
    sm = None
    kk = N
    for W, b, pw in ((W1, b1, pw1), (W2, b2, pw2), (W3, b3, pw3)):
        kk = int(math.ceil(0.5 * kk))
        f = W.shape[1]
        xw = _mm(h, W) if sm is None else _mm_scaled(h, sm, W)
        degp = _deg_sc(mb[:, 0], src_f, dst_f)
        y, ds, dis2 = _prep(degp.T, mb, xw)
        aggp = _gs(y, src_b, dst_b)
        h, s = _conv_score(aggp[:, :, :f], xw, ds, dis2, b, pw)
        sm, mb = _select(s, mb, pw, kk)

    xw = _mm_scaled(h, sm, W4)
    degp = _deg_sc(mb[:, 0], src_f, dst_f)
    y, ds, dis2 = _prep(degp.T, mb, xw)
    aggp = _gs(y, src_b, dst_b)
    out = _final(aggp[:, :, :16], xw, ds, dis2, mb, bt, b4, lw, lb)
    return out[:, 0]


# speedup vs baseline: 15.5678x; 1.0006x over previous
import functools
import math

import jax
import jax.numpy as jnp
from jax import lax
from jax.experimental import pallas as pl
from jax.experimental.pallas import tpu as pltpu
from jax.experimental.pallas import tpu_sc as plsc

N = 10000
NPAD = 10240
E = 320000
EPAD = 327680
B = 16
NW = 32
EBLK = 80
EW = 128
EHALF = EBLK // 2
RPT = NPAD // 16

_HI = lax.Precision.HIGHEST
_IMIN = -2147483648


def _dot(a, b, dims):
    return lax.dot_general(a, b, (dims, ((), ())), precision=_HI,
                           preferred_element_type=jnp.float32)


RB = 1280
NB = NPAD // RB


def _mm_body(h_ref, w_ref, o_ref):
    o_ref[...] = _dot(h_ref[...], w_ref[...], ((1,), (0,)))


def _mm(h, W):
    return pl.pallas_call(
        _mm_body,
        out_shape=jax.ShapeDtypeStruct((h.shape[0], W.shape[1]), jnp.float32),
    )(h, W)


def _mm_scaled_body(h_ref, sm_ref, w_ref, o_ref):
    o_ref[...] = _dot(sm_ref[...] * h_ref[...], w_ref[...], ((1,), (0,)))


def _mm_scaled(h, sm, W):
    return pl.pallas_call(
        _mm_scaled_body,
        out_shape=jax.ShapeDtypeStruct((h.shape[0], W.shape[1]), jnp.float32),
    )(h, sm, W)


def _prep_body(f, degp_ref, mb_ref, xw_ref, y_ref, ds_ref, dis2_ref):
    deg = 1.0 + jnp.sum(degp_ref[...], axis=1, keepdims=True)
    dis = lax.rsqrt(deg)
    ds = dis * mb_ref[:, 0:1]
    ds_ref[...] = ds
    dis2_ref[...] = dis * dis
    y_ref[:, :f] = ds * xw_ref[...]
    if f < 128:
        y_ref[:, f:] = jnp.zeros((NPAD, 128 - f), jnp.float32)


def _prep(degp_t, mb, xw):
    f = xw.shape[1]
    return pl.pallas_call(
        functools.partial(_prep_body, f),
        out_shape=(
            jax.ShapeDtypeStruct((NPAD, 128), jnp.float32),
            jax.ShapeDtypeStruct((NPAD, 1), jnp.float32),
            jax.ShapeDtypeStruct((NPAD, 1), jnp.float32),
        ),
    )(degp_t, mb, xw)


def _conv_out(aggp_ref, xw_ref, ds_ref, dis2_ref, b_ref):
    a = aggp_ref[...]
    agg = (ds_ref[...] * (a[0] + a[1]) + xw_ref[...] * dis2_ref[...]
           + b_ref[...])
    return jnp.maximum(agg, 0.0)


def _count(pred):
    return jnp.sum(jnp.where(pred, 1, 0))


def _conv_score_body(aggp_ref, xw_ref, ds_ref, dis2_ref, b_ref, pw_ref,
                     h_ref, s_ref):
    h1 = _conv_out(aggp_ref, xw_ref, ds_ref, dis2_ref, b_ref)
    h_ref[...] = h1
    s_ref[...] = _dot(h1, pw_ref[...], ((1,), (0,)))


def _conv_score(aggp, xw, ds, dis2, b, pw):
    f = xw.shape[1]
    return pl.pallas_call(
        _conv_score_body,
        grid=(NB,),
        in_specs=[
            pl.BlockSpec((2, RB, f), lambda i: (0, i, 0)),
            pl.BlockSpec((RB, f), lambda i: (i, 0)),
            pl.BlockSpec((RB, 1), lambda i: (i, 0)),
            pl.BlockSpec((RB, 1), lambda i: (i, 0)),
            pl.BlockSpec((1, f), lambda i: (0, 0)),
            pl.BlockSpec((f, 1), lambda i: (0, 0)),
        ],
        out_specs=[
            pl.BlockSpec((RB, f), lambda i: (i, 0)),
            pl.BlockSpec((RB, 1), lambda i: (i, 0)),
        ],
        out_shape=(
            jax.ShapeDtypeStruct((NPAD, f), jnp.float32),
            jax.ShapeDtypeStruct((NPAD, 1), jnp.float32),
        ),
    )(aggp, xw, ds, dis2, b.reshape(1, f), pw.reshape(f, 1))


def _select_body(k, s_ref, mb_ref, pw_ref, sm_ref, mb_out_ref):
    pw = pw_ref[...]
    pwn = jnp.sqrt(jnp.sum(pw * pw)) + 1e-16
    score = jnp.tanh(s_ref[...] / pwn)

    valid = mb_ref[:, 0:1] > 0.0
    bk = lax.bitcast_convert_type(score, jnp.int32)
    key = jnp.where(bk >= 0, bk, bk ^ jnp.int32(0x7FFFFFFF))
    key = jnp.where(valid, key, jnp.int32(_IMIN))

    nonneg = _count(key >= 0)
    lo = jnp.where(nonneg >= k, jnp.int32(0), jnp.int32(_IMIN))
    hi = jnp.where(nonneg >= k, jnp.int32(2147483647), jnp.int32(-1))

    def bis(_, lh):
        lo, hi = lh
        mid = lo + ((hi - lo) >> 1)
        down = _count(key > mid) < k
        return (jnp.where(down, lo, mid + 1), jnp.where(down, mid, hi))

    lo, _hi = lax.fori_loop(0, 31, bis, (lo, hi))
    t = lo
    need = k - _count(key > t)
    idx = lax.broadcasted_iota(jnp.int32, (NPAD, 1), 0)
    eq = key == t

    def bis2(_, lh):
        lo, hi = lh
        mid = lo + ((hi - lo) >> 1)
        down = _count(eq & (idx <= mid)) >= need
        return (jnp.where(down, lo, mid + 1), jnp.where(down, mid, hi))

    j, _ = lax.fori_loop(0, 14, bis2, (jnp.int32(0), jnp.int32(NPAD - 1)))
    sel = (key > t) | (eq & (idx <= j))
    mnew = jnp.where(sel, 1.0, 0.0)
    sm_ref[...] = score * mnew
    mb_out_ref[...] = jnp.broadcast_to(mnew, (NPAD, 16))


def _select(s, mb, pw, k):
    f = pw.shape[0]
    return pl.pallas_call(
        functools.partial(_select_body, k),
        out_shape=(
            jax.ShapeDtypeStruct((NPAD, 1), jnp.float32),
            jax.ShapeDtypeStruct((NPAD, 16), jnp.float32),
        ),
    )(s, mb, pw.reshape(f, 1))


def _final_body(aggp_ref, xw_ref, ds_ref, dis2_ref, mb_ref, bt_ref, b_ref,
                lw_ref, lb_ref, o_ref, acc_s, acc_c):
    i = pl.program_id(0)
    h1 = _conv_out(aggp_ref, xw_ref, ds_ref, dis2_ref, b_ref)
    mcol = mb_ref[:, 0:1]
    hm = h1 * mcol
    cols = lax.broadcasted_iota(jnp.int32, (RB, B), 1)
    oh = jnp.where((bt_ref[...] == cols) & (mcol > 0.0), 1.0, 0.0)
    bs = _dot(oh, hm, ((0,), (0,)))
    bc = _dot(oh, jnp.ones((RB, 1), jnp.float32), ((0,), (0,)))

    @pl.when(i == 0)
    def _():
        acc_s[...] = bs
        acc_c[...] = bc

    @pl.when(i != 0)
    def _():
        acc_s[...] += bs
        acc_c[...] += bc

    @pl.when(i == NB - 1)
    def _():
        mean = acc_s[...] / jnp.maximum(acc_c[...], 1.0)
        o_ref[...] = ((_dot(mean, lw_ref[...], ((1,), (0,))) + lb_ref[...])
                      * 100.0)


def _final(aggp, xw, ds, dis2, mb, bt, b, lw, lb):
    return pl.pallas_call(
        _final_body,
        grid=(NB,),
        in_specs=[
            pl.BlockSpec((2, RB, 16), lambda i: (0, i, 0)),
            pl.BlockSpec((RB, 16), lambda i: (i, 0)),
            pl.BlockSpec((RB, 1), lambda i: (i, 0)),
            pl.BlockSpec((RB, 1), lambda i: (i, 0)),
            pl.BlockSpec((RB, 16), lambda i: (i, 0)),
            pl.BlockSpec((RB, 1), lambda i: (i, 0)),
            pl.BlockSpec((1, 16), lambda i: (0, 0)),
            pl.BlockSpec((16, 1), lambda i: (0, 0)),
            pl.BlockSpec((1, 1), lambda i: (0, 0)),
        ],
        out_specs=pl.BlockSpec((B, 1), lambda i: (0, 0)),
        out_shape=jax.ShapeDtypeStruct((B, 1), jnp.float32),
        scratch_shapes=[
            pltpu.VMEM((B, 16), jnp.float32),
            pltpu.VMEM((B, 1), jnp.float32),
        ],
    )(aggp, xw, ds, dis2, mb, bt, b.reshape(1, 16), lw, lb.reshape(1, 1))


def _gs(y, src_b, dst_b):
    f = 128
    mesh = plsc.VectorSubcoreMesh(core_axis_name="c", subcore_axis_name="s")

    @functools.partial(
        pl.kernel,
        out_type=jax.ShapeDtypeStruct((2 * NPAD, f), jnp.float32),
        mesh=mesh,
        scratch_types=[
            pltpu.VMEM((EHALF, EW), jnp.int32),
            pltpu.VMEM((EHALF, EW), jnp.int32),
            pltpu.VMEM((EW, f), jnp.float32),
            pltpu.VMEM((EW, f), jnp.float32),
            pltpu.VMEM_SHARED((NPAD, f), jnp.float32),
            pltpu.SemaphoreType.DMA,
            pltpu.SemaphoreType.DMA,
        ],
    )
    def k(y_hbm, src_hbm, dst_hbm, out_hbm, src_v, dst_v, buf0, buf1,
          acc_sh, sem0, sem1):
        cid = lax.axis_index("c")
        sid = lax.axis_index("s")
        wid = cid * 16 + sid
        y_from = y_hbm

        @pl.loop(0, EW)
        def _(i):
            @pl.loop(0, f, step=16)
            def _(j):
                buf0[i, pl.ds(j, 16)] = jnp.zeros((16,), jnp.float32)

        @pl.loop(0, RPT, step=EW)
        def _(r):
            pltpu.sync_copy(buf0, acc_sh.at[pl.ds(sid * RPT + r, EW)])

        plsc.subcore_barrier()

        for half in range(2):
            pltpu.sync_copy(src_hbm.at[wid, pl.ds(half * EHALF, EHALF)],
                            src_v)
            pltpu.sync_copy(dst_hbm.at[wid, pl.ds(half * EHALF, EHALF)],
                            dst_v)
            pltpu.async_copy(y_from.at[src_v.at[0]], buf0, sem0)

            @pl.loop(0, EHALF, step=2)
            def _(i):
                pltpu.make_async_copy(y_from.at[src_v.at[i]], buf0,
                                      sem0).wait()
                pltpu.async_copy(y_from.at[src_v.at[i + 1]], buf1, sem1)
                pltpu.sync_copy(buf0, acc_sh.at[dst_v.at[i]], add=True)
                pltpu.make_async_copy(y_from.at[src_v.at[i + 1]], buf1,
                                      sem1).wait()

                @pl.when(i + 2 < EHALF)
                def _():
                    pltpu.async_copy(y_from.at[src_v.at[i + 2]], buf0, sem0)

                pltpu.sync_copy(buf1, acc_sh.at[dst_v.at[i + 1]], add=True)

        plsc.subcore_barrier()
        pltpu.sync_copy(acc_sh.at[pl.ds(sid * RPT, RPT)],
                        out_hbm.at[pl.ds(cid * NPAD + sid * RPT, RPT)])

    return k(y, src_b, dst_b).reshape(2, NPAD, f)


EPT = EPAD // NW


def _deg_sc(m1, src_f, dst_f):
    mesh = plsc.VectorSubcoreMesh(core_axis_name="c", subcore_axis_name="s")

    @functools.partial(
        pl.kernel,
        out_type=jax.ShapeDtypeStruct((NW, NPAD), jnp.float32),
        mesh=mesh,
        compiler_params=pltpu.CompilerParams(needs_layout_passes=False),
        scratch_types=[
            pltpu.VMEM((EPT,), jnp.int32),
            pltpu.VMEM((EPT,), jnp.int32),
            pltpu.VMEM((NPAD,), jnp.float32),
            pltpu.VMEM((NPAD,), jnp.float32),
        ],
    )
    def k(m_hbm, src_hbm, dst_hbm, out_hbm, src_v, dst_v, m_v, deg_v):
        cid = lax.axis_index("c")
        sid = lax.axis_index("s")
        wid = cid * 16 + sid

        @pl.loop(0, NPAD, step=16)
        def _(i):
            deg_v[pl.ds(i, 16)] = jnp.zeros((16,), jnp.float32)

        pltpu.sync_copy(m_hbm, m_v)
        pltpu.sync_copy(src_hbm.at[wid], src_v)
        pltpu.sync_copy(dst_hbm.at[wid], dst_v)

        @pl.loop(0, EPT, step=16)
        def _(i):
            sv = src_v[pl.ds(i, 16)]
            dv = dst_v[pl.ds(i, 16)]
            mv = plsc.load_gather(m_v, [sv])
            plsc.addupdate_scatter(deg_v, [dv], mv)

        pltpu.sync_copy(deg_v, out_hbm.at[wid])

    return k(m1, src_f, dst_f)


def kernel(x, edge_index, batch, W1, b1, pw1, W2, b2, pw2, W3, b3, pw3,
           W4, b4, lw, lb):
    xp = jnp.pad(x, ((0, NPAD - N), (0, 0)))
    src = jnp.pad(edge_index[0], (0, EPAD - E), constant_values=NPAD - 1)
    dst = jnp.pad(edge_index[1], (0, EPAD - E), constant_values=NPAD - 1)
    src_b = src.reshape(NW, EBLK, EW)
    dst_b = dst.reshape(NW, EBLK, EW)
    src_f = src.reshape(NW, EPT)
    dst_f = dst.reshape(NW, EPT)
    mb = jnp.pad(jnp.ones((N, 16), jnp.float32), ((0, NPAD - N), (0, 0)))
    bt = jnp.pad(batch, (0, NPAD - N)).reshape(NPAD, 1)

    h = xp
    sm = None
    kk = N
    for W, b, pw in ((W1, b1, pw1), (W2, b2, pw2), (W3, b3, pw3)):
        kk = int(math.ceil(0.5 * kk))
        f = W.shape[1]
        xw = _mm(h, W) if sm is None else _mm_scaled(h, sm, W)
        degp = _deg_sc(mb[:, 0], src_f, dst_f)
        y, ds, dis2 = _prep(degp.T, mb, xw)
        aggp = _gs(y, src_b, dst_b)
        h, s = _conv_score(aggp[:, :, :f], xw, ds, dis2, b, pw)
        sm, mb = _select(s, mb, pw, kk)

    xw = _mm_scaled(h, sm, W4)
    degp = _deg_sc(mb[:, 0], src_f, dst_f)
    y, ds, dis2 = _prep(degp.T, mb, xw)
    aggp = _gs(y, src_b, dst_b)
    out = _final(aggp[:, :, :16], xw, ds, dis2, mb, bt, b4, lw, lb)
    return out[:, 0]


# speedup vs baseline: 15.6356x; 1.0043x over previous
import functools
import math

import jax
import jax.numpy as jnp
from jax import lax
from jax.experimental import pallas as pl
from jax.experimental.pallas import tpu as pltpu
from jax.experimental.pallas import tpu_sc as plsc

N = 10000
NPAD = 10240
E = 320000
EPAD = 327680
B = 16
NW = 32
EBLK = 80
EW = 128
EHALF = EBLK // 2
RPT = NPAD // 16

_HI = lax.Precision.HIGHEST
_IMIN = -2147483648


def _dot(a, b, dims):
    return lax.dot_general(a, b, (dims, ((), ())), precision=_HI,
                           preferred_element_type=jnp.float32)


RB = 1280
NB = NPAD // RB


def _mm_body(h_ref, w_ref, o_ref):
    o_ref[...] = _dot(h_ref[...], w_ref[...], ((1,), (0,)))


def _mm(h, W):
    return pl.pallas_call(
        _mm_body,
        out_shape=jax.ShapeDtypeStruct((h.shape[0], W.shape[1]), jnp.float32),
    )(h, W)


def _mm_scaled_body(h_ref, sm_ref, w_ref, o_ref):
    o_ref[...] = _dot(sm_ref[...] * h_ref[...], w_ref[...], ((1,), (0,)))


def _mm_scaled(h, sm, W):
    return pl.pallas_call(
        _mm_scaled_body,
        out_shape=jax.ShapeDtypeStruct((h.shape[0], W.shape[1]), jnp.float32),
    )(h, sm, W)


def _prep_body(f, degp_ref, mb_ref, xw_ref, y_ref, ds_ref, dis2_ref):
    deg = 1.0 + jnp.sum(degp_ref[...], axis=1, keepdims=True)
    dis = lax.rsqrt(deg)
    ds = dis * mb_ref[:, 0:1]
    ds_ref[...] = ds
    dis2_ref[...] = dis * dis
    y_ref[:, :f] = ds * xw_ref[...]
    if f < 128:
        y_ref[:, f:] = jnp.zeros((NPAD, 128 - f), jnp.float32)


def _prep(degp_t, mb, xw):
    f = xw.shape[1]
    return pl.pallas_call(
        functools.partial(_prep_body, f),
        out_shape=(
            jax.ShapeDtypeStruct((NPAD, 128), jnp.float32),
            jax.ShapeDtypeStruct((NPAD, 1), jnp.float32),
            jax.ShapeDtypeStruct((NPAD, 1), jnp.float32),
        ),
    )(degp_t, mb, xw)


def _conv_out(aggp_ref, xw_ref, ds_ref, dis2_ref, b_ref):
    a = aggp_ref[...]
    agg = (ds_ref[...] * (a[0] + a[1]) + xw_ref[...] * dis2_ref[...]
           + b_ref[...])
    return jnp.maximum(agg, 0.0)


def _count(pred):
    return jnp.sum(jnp.where(pred, 1, 0))


def _conv_score_body(aggp_ref, xw_ref, ds_ref, dis2_ref, b_ref, pw_ref,
                     h_ref, s_ref):
    h1 = _conv_out(aggp_ref, xw_ref, ds_ref, dis2_ref, b_ref)
    h_ref[...] = h1
    s_ref[...] = _dot(h1, pw_ref[...], ((1,), (0,)))


def _conv_score(aggp, xw, ds, dis2, b, pw):
    f = xw.shape[1]
    return pl.pallas_call(
        _conv_score_body,
        grid=(NB,),
        in_specs=[
            pl.BlockSpec((2, RB, f), lambda i: (0, i, 0)),
            pl.BlockSpec((RB, f), lambda i: (i, 0)),
            pl.BlockSpec((RB, 1), lambda i: (i, 0)),
            pl.BlockSpec((RB, 1), lambda i: (i, 0)),
            pl.BlockSpec((1, f), lambda i: (0, 0)),
            pl.BlockSpec((f, 1), lambda i: (0, 0)),
        ],
        out_specs=[
            pl.BlockSpec((RB, f), lambda i: (i, 0)),
            pl.BlockSpec((RB, 1), lambda i: (i, 0)),
        ],
        out_shape=(
            jax.ShapeDtypeStruct((NPAD, f), jnp.float32),
            jax.ShapeDtypeStruct((NPAD, 1), jnp.float32),
        ),
    )(aggp, xw, ds, dis2, b.reshape(1, f), pw.reshape(f, 1))


def _select_body(k, s_ref, mb_ref, pw_ref, sm_ref, mb_out_ref):
    pw = pw_ref[...]
    pwn = jnp.sqrt(jnp.sum(pw * pw)) + 1e-16
    score = jnp.tanh(s_ref[...] / pwn)

    valid = mb_ref[:, 0:1] > 0.0
    bk = lax.bitcast_convert_type(score, jnp.int32)
    key = jnp.where(bk >= 0, bk, bk ^ jnp.int32(0x7FFFFFFF))
    key = jnp.where(valid, key, jnp.int32(_IMIN))

    nonneg = _count(key >= 0)
    lo = jnp.where(nonneg >= k, jnp.int32(0), jnp.int32(_IMIN))
    hi = jnp.where(nonneg >= k, jnp.int32(2147483647), jnp.int32(-1))

    def bis(_, lh):
        lo, hi = lh
        mid = lo + ((hi - lo) >> 1)
        down = _count(key > mid) < k
        return (jnp.where(down, lo, mid + 1), jnp.where(down, mid, hi))

    lo, _hi = lax.fori_loop(0, 31, bis, (lo, hi))
    t = lo
    need = k - _count(key > t)
    idx = lax.broadcasted_iota(jnp.int32, (NPAD, 1), 0)
    eq = key == t

    def bis2(_, lh):
        lo, hi = lh
        mid = lo + ((hi - lo) >> 1)
        down = _count(eq & (idx <= mid)) >= need
        return (jnp.where(down, lo, mid + 1), jnp.where(down, mid, hi))

    j, _ = lax.fori_loop(0, 14, bis2, (jnp.int32(0), jnp.int32(NPAD - 1)))
    sel = (key > t) | (eq & (idx <= j))
    mnew = jnp.where(sel, 1.0, 0.0)
    sm_ref[...] = score * mnew
    mb_out_ref[...] = jnp.broadcast_to(mnew, (NPAD, 16))


def _select(s, mb, pw, k):
    f = pw.shape[0]
    return pl.pallas_call(
        functools.partial(_select_body, k),
        out_shape=(
            jax.ShapeDtypeStruct((NPAD, 1), jnp.float32),
            jax.ShapeDtypeStruct((NPAD, 16), jnp.float32),
        ),
    )(s, mb, pw.reshape(f, 1))


def _final_body(aggp_ref, xw_ref, ds_ref, dis2_ref, mb_ref, bt_ref, b_ref,
                lw_ref, lb_ref, o_ref, acc_s, acc_c):
    i = pl.program_id(0)
    h1 = _conv_out(aggp_ref, xw_ref, ds_ref, dis2_ref, b_ref)
    mcol = mb_ref[:, 0:1]
    hm = h1 * mcol
    cols = lax.broadcasted_iota(jnp.int32, (RB, B), 1)
    oh = jnp.where((bt_ref[...] == cols) & (mcol > 0.0), 1.0, 0.0)
    bs = _dot(oh, hm, ((0,), (0,)))
    bc = _dot(oh, jnp.ones((RB, 1), jnp.float32), ((0,), (0,)))

    @pl.when(i == 0)
    def _():
        acc_s[...] = bs
        acc_c[...] = bc

    @pl.when(i != 0)
    def _():
        acc_s[...] += bs
        acc_c[...] += bc

    @pl.when(i == NB - 1)
    def _():
        mean = acc_s[...] / jnp.maximum(acc_c[...], 1.0)
        o_ref[...] = ((_dot(mean, lw_ref[...], ((1,), (0,))) + lb_ref[...])
                      * 100.0)


def _final(aggp, xw, ds, dis2, mb, bt, b, lw, lb):
    return pl.pallas_call(
        _final_body,
        grid=(NB,),
        in_specs=[
            pl.BlockSpec((2, RB, 16), lambda i: (0, i, 0)),
            pl.BlockSpec((RB, 16), lambda i: (i, 0)),
            pl.BlockSpec((RB, 1), lambda i: (i, 0)),
            pl.BlockSpec((RB, 1), lambda i: (i, 0)),
            pl.BlockSpec((RB, 16), lambda i: (i, 0)),
            pl.BlockSpec((RB, 1), lambda i: (i, 0)),
            pl.BlockSpec((1, 16), lambda i: (0, 0)),
            pl.BlockSpec((16, 1), lambda i: (0, 0)),
            pl.BlockSpec((1, 1), lambda i: (0, 0)),
        ],
        out_specs=pl.BlockSpec((B, 1), lambda i: (0, 0)),
        out_shape=jax.ShapeDtypeStruct((B, 1), jnp.float32),
        scratch_shapes=[
            pltpu.VMEM((B, 16), jnp.float32),
            pltpu.VMEM((B, 1), jnp.float32),
        ],
    )(aggp, xw, ds, dis2, mb, bt, b.reshape(1, 16), lw, lb.reshape(1, 1))


def _gs(y, src_b, dst_b):
    f = 128
    mesh = plsc.VectorSubcoreMesh(core_axis_name="c", subcore_axis_name="s")

    @functools.partial(
        pl.kernel,
        out_type=jax.ShapeDtypeStruct((2 * NPAD, f), jnp.float32),
        mesh=mesh,
        scratch_types=[
            pltpu.VMEM((EHALF, EW), jnp.int32),
            pltpu.VMEM((EHALF, EW), jnp.int32),
            pltpu.VMEM((EW, f), jnp.float32),
            pltpu.VMEM((EW, f), jnp.float32),
            pltpu.VMEM_SHARED((NPAD, f), jnp.float32),
            pltpu.SemaphoreType.DMA,
            pltpu.SemaphoreType.DMA,
            pltpu.SemaphoreType.DMA,
            pltpu.SemaphoreType.DMA,
        ],
    )
    def k(y_hbm, src_hbm, dst_hbm, out_hbm, src_v, dst_v, buf0, buf1,
          acc_sh, sem0, sem1, ssem0, ssem1):
        cid = lax.axis_index("c")
        sid = lax.axis_index("s")
        wid = cid * 16 + sid
        y_from = y_hbm

        @pl.loop(0, EW)
        def _(i):
            @pl.loop(0, f, step=16)
            def _(j):
                buf0[i, pl.ds(j, 16)] = jnp.zeros((16,), jnp.float32)

        @pl.loop(0, RPT, step=EW)
        def _(r):
            pltpu.sync_copy(buf0, acc_sh.at[pl.ds(sid * RPT + r, EW)])

        plsc.subcore_barrier()

        for half in range(2):
            pltpu.sync_copy(src_hbm.at[wid, pl.ds(half * EHALF, EHALF)],
                            src_v)
            pltpu.sync_copy(dst_hbm.at[wid, pl.ds(half * EHALF, EHALF)],
                            dst_v)
            pltpu.async_copy(y_from.at[src_v.at[0]], buf0, sem0)
            pltpu.async_copy(y_from.at[src_v.at[1]], buf1, sem1)

            @pl.loop(0, EHALF, step=2)
            def _(i):
                pltpu.make_async_copy(y_from.at[src_v.at[i]], buf0,
                                      sem0).wait()
                pltpu.async_copy(buf0, acc_sh.at[dst_v.at[i]], ssem0,
                                 add=True)
                pltpu.make_async_copy(y_from.at[src_v.at[i + 1]], buf1,
                                      sem1).wait()
                pltpu.async_copy(buf1, acc_sh.at[dst_v.at[i + 1]], ssem1,
                                 add=True)
                pltpu.make_async_copy(buf0, acc_sh.at[dst_v.at[i]],
                                      ssem0).wait()

                @pl.when(i + 2 < EHALF)
                def _():
                    pltpu.async_copy(y_from.at[src_v.at[i + 2]], buf0, sem0)

                pltpu.make_async_copy(buf1, acc_sh.at[dst_v.at[i + 1]],
                                      ssem1).wait()

                @pl.when(i + 3 < EHALF)
                def _():
                    pltpu.async_copy(y_from.at[src_v.at[i + 3]], buf1, sem1)

        plsc.subcore_barrier()
        pltpu.sync_copy(acc_sh.at[pl.ds(sid * RPT, RPT)],
                        out_hbm.at[pl.ds(cid * NPAD + sid * RPT, RPT)])

    return k(y, src_b, dst_b).reshape(2, NPAD, f)


EPT = EPAD // NW


def _deg_sc(m1, src_f, dst_f):
    mesh = plsc.VectorSubcoreMesh(core_axis_name="c", subcore_axis_name="s")

    @functools.partial(
        pl.kernel,
        out_type=jax.ShapeDtypeStruct((NW, NPAD), jnp.float32),
        mesh=mesh,
        compiler_params=pltpu.CompilerParams(needs_layout_passes=False),
        scratch_types=[
            pltpu.VMEM((EPT,), jnp.int32),
            pltpu.VMEM((EPT,), jnp.int32),
            pltpu.VMEM((NPAD,), jnp.float32),
            pltpu.VMEM((NPAD,), jnp.float32),
        ],
    )
    def k(m_hbm, src_hbm, dst_hbm, out_hbm, src_v, dst_v, m_v, deg_v):
        cid = lax.axis_index("c")
        sid = lax.axis_index("s")
        wid = cid * 16 + sid

        @pl.loop(0, NPAD, step=16)
        def _(i):
            deg_v[pl.ds(i, 16)] = jnp.zeros((16,), jnp.float32)

        pltpu.sync_copy(m_hbm, m_v)
        pltpu.sync_copy(src_hbm.at[wid], src_v)
        pltpu.sync_copy(dst_hbm.at[wid], dst_v)

        @pl.loop(0, EPT, step=16)
        def _(i):
            sv = src_v[pl.ds(i, 16)]
            dv = dst_v[pl.ds(i, 16)]
            mv = plsc.load_gather(m_v, [sv])
            plsc.addupdate_scatter(deg_v, [dv], mv)

        pltpu.sync_copy(deg_v, out_hbm.at[wid])

    return k(m1, src_f, dst_f)


def kernel(x, edge_index, batch, W1, b1, pw1, W2, b2, pw2, W3, b3, pw3,
           W4, b4, lw, lb):
    xp = jnp.pad(x, ((0, NPAD - N), (0, 0)))
    src = jnp.pad(edge_index[0], (0, EPAD - E), constant_values=NPAD - 1)
    dst = jnp.pad(edge_index[1], (0, EPAD - E), constant_values=NPAD - 1)
    src_b = src.reshape(NW, EBLK, EW)
    dst_b = dst.reshape(NW, EBLK, EW)
    src_f = src.reshape(NW, EPT)
    dst_f = dst.reshape(NW, EPT)
    mb = jnp.pad(jnp.ones((N, 16), jnp.float32), ((0, NPAD - N), (0, 0)))
    bt = jnp.pad(batch, (0, NPAD - N)).reshape(NPAD, 1)

    h = xp
    sm = None
    kk = N
    for W, b, pw in ((W1, b1, pw1), (W2, b2, pw2), (W3, b3, pw3)):
        kk = int(math.ceil(0.5 * kk))
        f = W.shape[1]
        xw = _mm(h, W) if sm is None else _mm_scaled(h, sm, W)
        degp = _deg_sc(mb[:, 0], src_f, dst_f)
        y, ds, dis2 = _prep(degp.T, mb, xw)
        aggp = _gs(y, src_b, dst_b)
        h, s = _conv_score(aggp[:, :, :f], xw, ds, dis2, b, pw)
        sm, mb = _select(s, mb, pw, kk)

    xw = _mm_scaled(h, sm, W4)
    degp = _deg_sc(mb[:, 0], src_f, dst_f)
    y, ds, dis2 = _prep(degp.T, mb, xw)
    aggp = _gs(y, src_b, dst_b)
    out = _final(aggp[:, :, :16], xw, ds, dis2, mb, bt, b4, lw, lb)
    return out[:, 0]
